# baseline (speedup 1.0000x reference)
import functools
import math

import jax
import jax.numpy as jnp
from jax import lax
from jax.experimental import pallas as pl
from jax.experimental.pallas import tpu as pltpu
from jax.experimental.pallas import tpu_sc as plsc

_EMBED = 128
_SCALE = math.sqrt(float(_EMBED))


@functools.lru_cache(maxsize=None)
def _make_gather(N, D):
    info = plsc.get_sparse_core_info()
    NC, NS = info.num_cores, info.num_subcores
    NW = NC * NS
    assert N % NW == 0
    per_w = N // NW
    K = 400
    NB = 2
    assert per_w % K == 0
    n_chunks = per_w // K
    assert n_chunks >= 3 * NB and (n_chunks - 2 * NB) % NB == 0
    mesh = plsc.VectorSubcoreMesh(core_axis_name="c", subcore_axis_name="s")

    @functools.partial(
        pl.kernel,
        mesh=mesh,
        out_type=jax.ShapeDtypeStruct((N, D), jnp.float32),
        scratch_types=(
            [pltpu.VMEM((K,), jnp.int32) for _ in range(NB)]
            + [pltpu.VMEM((K, D), jnp.float32) for _ in range(NB)]
            + [pltpu.SemaphoreType.DMA for _ in range(3 * NB)]
        ),
    )
    def gather_kernel(idx_hbm, table_hbm, out_hbm, *scratch):
        idx_v = scratch[:NB]
        rows_v = scratch[NB:2 * NB]
        sem_i = scratch[2 * NB:3 * NB]
        sem_g = scratch[3 * NB:4 * NB]
        sem_o = scratch[4 * NB:5 * NB]
        wid = lax.axis_index("s") * NC + lax.axis_index("c")
        base = wid * per_w

        def idx_cp(c, b):
            off = pl.multiple_of(base + c * K, 8)
            return pltpu.make_async_copy(
                idx_hbm.at[pl.ds(off, K)], idx_v[b], sem_i[b])

        def out_cp(c, b):
            off = pl.multiple_of(base + c * K, 8)
            return pltpu.make_async_copy(
                rows_v[b], out_hbm.at[pl.ds(off, K)], sem_o[b])

        def gather_cp(b):
            return pltpu.make_async_copy(
                table_hbm.at[idx_v[b]], rows_v[b], sem_g[b])

        RR = 4

        def scale_rows(b):
            ref = rows_v[b]

            def sbody(r0, carry):
                for rr in range(RR):
                    r = r0 * RR + rr
                    for j in range(D // 16):
                        sl = pl.ds(16 * j, 16)
                        ref[r, sl] = ref[r, sl] * _SCALE
                return carry

            lax.fori_loop(0, K // RR, sbody, 0)

        for c in range(NB):
            idx_cp(c, c).start()
        for c in range(NB):
            b = c
            idx_cp(c, b).wait()
            gather_cp(b).start()
            if c >= 1:
                b1 = c - 1
                gather_cp(b1).wait()
                scale_rows(b1)
                out_cp(c - 1, b1).start()
                idx_cp(c + NB - 1, b1).start()

        def body(p, carry):
            for j in range(NB):
                c = NB * p + NB + j
                b = j
                b1 = (j - 1) % NB
                idx_cp(c, b).wait()
                out_cp(c - NB, b).wait()
                gather_cp(b).start()
                gather_cp(b1).wait()
                scale_rows(b1)
                out_cp(c - 1, b1).start()
                idx_cp(c + NB - 1, b1).start()
            return carry

        lax.fori_loop(0, (n_chunks - 2 * NB) // NB, body, 0)

        for c in range(n_chunks - NB, n_chunks):
            b = c % NB
            b1 = (c - 1) % NB
            idx_cp(c, b).wait()
            out_cp(c - NB, b).wait()
            gather_cp(b).start()
            gather_cp(b1).wait()
            scale_rows(b1)
            out_cp(c - 1, b1).start()
            if c + NB - 1 < n_chunks:
                idx_cp(c + NB - 1, b1).start()

        bl = (n_chunks - 1) % NB
        gather_cp(bl).wait()
        scale_rows(bl)
        out_cp(n_chunks - 1, bl).start()
        for c in range(n_chunks - NB, n_chunks):
            out_cp(c, c % NB).wait()

    return gather_kernel


def kernel(x, table):
    B, L = x.shape
    V, D = table.shape
    N = B * L
    idx = x.reshape(N).astype(jnp.int32)
    out = _make_gather(N, D)(idx, table)
    return out.reshape(B, L, D)


# speedup vs baseline: 1.0061x; 1.0061x over previous
import functools
import math

import jax
import jax.numpy as jnp
from jax import lax
from jax.experimental import pallas as pl
from jax.experimental.pallas import tpu as pltpu
from jax.experimental.pallas import tpu_sc as plsc

_EMBED = 128
_SCALE = math.sqrt(float(_EMBED))


@functools.lru_cache(maxsize=None)
def _make_gather(N, D):
    info = plsc.get_sparse_core_info()
    NC, NS = info.num_cores, info.num_subcores
    NW = NC * NS
    assert N % NW == 0
    per_w = N // NW
    K = 160
    NB = 5
    assert per_w % K == 0
    n_chunks = per_w // K
    assert n_chunks >= 3 * NB and (n_chunks - 2 * NB) % NB == 0
    mesh = plsc.VectorSubcoreMesh(core_axis_name="c", subcore_axis_name="s")

    @functools.partial(
        pl.kernel,
        mesh=mesh,
        out_type=jax.ShapeDtypeStruct((N, D), jnp.float32),
        scratch_types=(
            [pltpu.VMEM((K,), jnp.int32) for _ in range(NB)]
            + [pltpu.VMEM((K, D), jnp.float32) for _ in range(NB)]
            + [pltpu.SemaphoreType.DMA for _ in range(3 * NB)]
        ),
    )
    def gather_kernel(idx_hbm, table_hbm, out_hbm, *scratch):
        idx_v = scratch[:NB]
        rows_v = scratch[NB:2 * NB]
        sem_i = scratch[2 * NB:3 * NB]
        sem_g = scratch[3 * NB:4 * NB]
        sem_o = scratch[4 * NB:5 * NB]
        wid = lax.axis_index("s") * NC + lax.axis_index("c")
        base = wid * per_w

        def idx_cp(c, b):
            off = pl.multiple_of(base + c * K, 8)
            return pltpu.make_async_copy(
                idx_hbm.at[pl.ds(off, K)], idx_v[b], sem_i[b])

        def out_cp(c, b):
            off = pl.multiple_of(base + c * K, 8)
            return pltpu.make_async_copy(
                rows_v[b], out_hbm.at[pl.ds(off, K)], sem_o[b])

        def gather_cp(b):
            return pltpu.make_async_copy(
                table_hbm.at[idx_v[b]], rows_v[b], sem_g[b])

        RR = 4

        def scale_rows(b):
            ref = rows_v[b]

            def sbody(r0, carry):
                for rr in range(RR):
                    r = r0 * RR + rr
                    for j in range(D // 16):
                        sl = pl.ds(16 * j, 16)
                        ref[r, sl] = ref[r, sl] * _SCALE
                return carry

            lax.fori_loop(0, K // RR, sbody, 0)

        for c in range(NB):
            idx_cp(c, c).start()
        for c in range(NB):
            b = c
            idx_cp(c, b).wait()
            gather_cp(b).start()
            if c >= 1:
                b1 = c - 1
                gather_cp(b1).wait()
                scale_rows(b1)
                out_cp(c - 1, b1).start()
                idx_cp(c + NB - 1, b1).start()

        def body(p, carry):
            for j in range(NB):
                c = NB * p + NB + j
                b = j
                b1 = (j - 1) % NB
                idx_cp(c, b).wait()
                out_cp(c - NB, b).wait()
                gather_cp(b).start()
                gather_cp(b1).wait()
                scale_rows(b1)
                out_cp(c - 1, b1).start()
                idx_cp(c + NB - 1, b1).start()
            return carry

        lax.fori_loop(0, (n_chunks - 2 * NB) // NB, body, 0)

        for c in range(n_chunks - NB, n_chunks):
            b = c % NB
            b1 = (c - 1) % NB
            idx_cp(c, b).wait()
            out_cp(c - NB, b).wait()
            gather_cp(b).start()
            gather_cp(b1).wait()
            scale_rows(b1)
            out_cp(c - 1, b1).start()
            if c + NB - 1 < n_chunks:
                idx_cp(c + NB - 1, b1).start()

        bl = (n_chunks - 1) % NB
        gather_cp(bl).wait()
        scale_rows(bl)
        out_cp(n_chunks - 1, bl).start()
        for c in range(n_chunks - NB, n_chunks):
            out_cp(c, c % NB).wait()

    return gather_kernel


def kernel(x, table):
    B, L = x.shape
    V, D = table.shape
    N = B * L
    idx = x.reshape(N).astype(jnp.int32)
    out = _make_gather(N, D)(idx, table)
    return out.reshape(B, L, D)
